# baseline (speedup 1.0000x reference)
import functools

import jax
import jax.numpy as jnp
from jax import lax
from jax.experimental import pallas as pl
from jax.experimental.pallas import tpu as pltpu
from jax.experimental.pallas import tpu_sc as plsc

B = 16384
L = 200
D = 32
NV = 1000000
NW = 32
BPW = B // NW
CB = 4
NCHUNKS = BPW // CB

TBC = 2048
TGRID = (NV + TBC - 1) // TBC
VPAD = TGRID * TBC

_mesh = plsc.VectorSubcoreMesh(core_axis_name="c", subcore_axis_name="s")


@functools.partial(
    pl.kernel,
    mesh=_mesh,
    out_type=jax.ShapeDtypeStruct((B, D), jnp.float32),
    scratch_types=[
        pltpu.VMEM((2, 2 * CB, 128), jnp.int32),
        pltpu.VMEM((2, CB * L, D), jnp.float32),
        pltpu.VMEM((BPW, D), jnp.float32),
        pltpu.SemaphoreType.DMA((2,)),
    ],
    compiler_params=pltpu.CompilerParams(use_tc_tiling_on_sc=False),
)
def _embbag(idx_hbm, table_hbm, out_hbm, idx_v, rows_v, out_v, sems):
    wid = lax.axis_index("s") * 2 + lax.axis_index("c")
    bag0 = wid * BPW
    scale = jnp.full((16,), 1.0 / L, jnp.float32)

    def fire(c, p):
        bb = bag0 + c * CB
        pltpu.sync_copy(idx_hbm.at[pl.ds(2 * bb, 2 * CB)], idx_v.at[p])
        for k in range(CB):
            pltpu.async_copy(
                table_hbm.at[idx_v.at[p, 2 * k]],
                rows_v.at[p, pl.ds(k * L, 128)],
                sems.at[p],
            )
            pltpu.async_copy(
                table_hbm.at[idx_v.at[p, 2 * k + 1, pl.ds(0, L - 128)]],
                rows_v.at[p, pl.ds(k * L + 128, L - 128)],
                sems.at[p],
            )

    def wait(p):
        pltpu.make_async_copy(
            table_hbm.at[pl.ds(0, CB * L)], rows_v.at[p], sems.at[p]
        ).wait()

    def reduce(c, p):
        for k in range(CB):
            base = k * L

            def red(j, accs):
                acc = list(accs)
                r = base + j * 8
                for u in range(8):
                    acc[u % 4] = acc[u % 4] + rows_v[p, r + u, 0:16]
                    acc[4 + u % 4] = acc[4 + u % 4] + rows_v[p, r + u, 16:32]
                return tuple(acc)

            z = jnp.zeros((16,), jnp.float32)
            accs = lax.fori_loop(0, L // 8, red, (z,) * 8)
            a_lo = (accs[0] + accs[1]) + (accs[2] + accs[3])
            a_hi = (accs[4] + accs[5]) + (accs[6] + accs[7])
            slot = c * CB + k
            out_v[slot, 0:16] = a_lo * scale
            out_v[slot, 16:32] = a_hi * scale

    fire(0, 0)

    def body(g, _):
        c0 = g * 2
        fire(c0 + 1, 1)
        wait(0)
        reduce(c0, 0)
        fire(c0 + 2, 0)
        wait(1)
        reduce(c0 + 1, 1)
        return 0

    lax.fori_loop(0, NCHUNKS // 2 - 1, body, 0)
    c0 = NCHUNKS - 2
    fire(c0 + 1, 1)
    wait(0)
    reduce(c0, 0)
    wait(1)
    reduce(c0 + 1, 1)

    pltpu.sync_copy(out_v, out_hbm.at[pl.ds(bag0, BPW)])


def _selector(q):
    r = lax.broadcasted_iota(jnp.int32, (32, 128), 0)
    c = lax.broadcasted_iota(jnp.int32, (32, 128), 1)
    return (c == 32 * q + r).astype(jnp.float32)


def _tr_table_body(w_ref, out_ref):
    x = w_ref[...]
    acc = jnp.zeros((TBC // 4, 128), jnp.float32)
    for q in range(4):
        xq = x[:, q * (TBC // 4):(q + 1) * (TBC // 4)]
        acc = acc + lax.dot_general(
            xq, _selector(q), (((0,), (0,)), ((), ())),
            preferred_element_type=jnp.float32,
            precision=lax.Precision.HIGHEST)
    out_ref[...] = acc


_tr_table = pl.pallas_call(
    _tr_table_body,
    grid=(TGRID,),
    in_specs=[pl.BlockSpec((32, TBC), lambda i: (0, i))],
    out_specs=pl.BlockSpec((TBC // 4, 128), lambda i: (i, 0)),
    out_shape=jax.ShapeDtypeStruct((VPAD * 32 // 128, 128), jnp.float32),
)

IBR = 2048


def _remap(bv):
    hi = (bv >> 11) << 11
    r = (bv & 511) << 2
    q = (bv >> 9) & 3
    return hi | r | q


def _tr_idx_body(x_ref, o_ref):
    X = x_ref[...]
    A = _remap(X[0:128, :].T)
    Bp = jnp.concatenate(
        [_remap(X[128:L, :].T), jnp.zeros((IBR, 128 - (L - 128)), jnp.int32)],
        axis=1)
    o_ref[...] = jnp.stack([A, Bp], axis=1).reshape(2 * IBR, 128)


_tr_idx = pl.pallas_call(
    _tr_idx_body,
    grid=(B // IBR,),
    in_specs=[pl.BlockSpec((L, IBR), lambda i: (0, i))],
    out_specs=pl.BlockSpec((2 * IBR, 128), lambda i: (i, 0)),
    out_shape=jax.ShapeDtypeStruct((2 * B, 128), jnp.int32),
)


def kernel(inlets, weight):
    wlin = _tr_table(weight.T)
    idx = _tr_idx(inlets[0].T)
    return _embbag(idx, wlin.reshape(VPAD, D))


# speedup vs baseline: 1.1721x; 1.1721x over previous
import functools

import jax
import jax.numpy as jnp
from jax import lax
from jax.experimental import pallas as pl
from jax.experimental.pallas import tpu as pltpu
from jax.experimental.pallas import tpu_sc as plsc

B = 16384
L = 200
D = 32
NV = 1000000
NW = 32
BPW = B // NW
CB = 4
NCHUNKS = BPW // CB

TBC = 2048
TGRID = (NV + TBC - 1) // TBC
VPAD = TGRID * TBC

_mesh = plsc.VectorSubcoreMesh(core_axis_name="c", subcore_axis_name="s")


@functools.partial(
    pl.kernel,
    mesh=_mesh,
    out_type=jax.ShapeDtypeStruct((B, D), jnp.float32),
    scratch_types=[
        pltpu.VMEM((2, 2 * CB, 128), jnp.int32),
        pltpu.VMEM((2, CB * L, D), jnp.float32),
        pltpu.VMEM((BPW, D), jnp.float32),
        pltpu.SemaphoreType.DMA((2,)),
    ],
    compiler_params=pltpu.CompilerParams(use_tc_tiling_on_sc=False),
)
def _embbag(idx_hbm, table_hbm, out_hbm, idx_v, rows_v, out_v, sems):
    wid = lax.axis_index("s") * 2 + lax.axis_index("c")
    bag0 = wid * BPW
    scale = jnp.full((16,), 1.0 / L, jnp.float32)

    def fire(c, p):
        bb = bag0 + c * CB
        pltpu.sync_copy(idx_hbm.at[pl.ds(2 * bb, 2 * CB)], idx_v.at[p])
        for k in range(CB):
            pltpu.async_copy(
                table_hbm.at[idx_v.at[p, 2 * k]],
                rows_v.at[p, pl.ds(k * L, 128)],
                sems.at[p],
            )
            pltpu.async_copy(
                table_hbm.at[idx_v.at[p, 2 * k + 1, pl.ds(0, L - 128)]],
                rows_v.at[p, pl.ds(k * L + 128, L - 128)],
                sems.at[p],
            )

    def wait(p):
        pltpu.make_async_copy(
            table_hbm.at[pl.ds(0, CB * L)], rows_v.at[p], sems.at[p]
        ).wait()

    def reduce(c, p):
        for k in range(CB):
            base = k * L

            def red(j, accs):
                acc = list(accs)
                r = base + j * 8
                for u in range(8):
                    acc[u % 4] = acc[u % 4] + rows_v[p, r + u, 0:16]
                    acc[4 + u % 4] = acc[4 + u % 4] + rows_v[p, r + u, 16:32]
                return tuple(acc)

            z = jnp.zeros((16,), jnp.float32)
            accs = lax.fori_loop(0, L // 8, red, (z,) * 8)
            a_lo = (accs[0] + accs[1]) + (accs[2] + accs[3])
            a_hi = (accs[4] + accs[5]) + (accs[6] + accs[7])
            slot = c * CB + k
            out_v[slot, 0:16] = a_lo * scale
            out_v[slot, 16:32] = a_hi * scale

    fire(0, 0)

    def body(g, _):
        c0 = g * 2
        fire(c0 + 1, 1)
        wait(0)
        reduce(c0, 0)
        fire(c0 + 2, 0)
        wait(1)
        reduce(c0 + 1, 1)
        return 0

    lax.fori_loop(0, NCHUNKS // 2 - 1, body, 0)
    c0 = NCHUNKS - 2
    fire(c0 + 1, 1)
    wait(0)
    reduce(c0, 0)
    wait(1)
    reduce(c0 + 1, 1)

    pltpu.sync_copy(out_v, out_hbm.at[pl.ds(bag0, BPW)])


def _tr_table_body(w_ref, out_ref):
    out_ref[:, 0:32] = w_ref[...].T


_tr_table = pl.pallas_call(
    _tr_table_body,
    grid=(TGRID,),
    in_specs=[pl.BlockSpec((32, TBC), lambda i: (0, i))],
    out_specs=pl.BlockSpec((TBC, 128), lambda i: (i, 0)),
    out_shape=jax.ShapeDtypeStruct((VPAD, 128), jnp.float32),
)

IBR = 2048


def _remap(bv):
    return bv << 2


def _tr_idx_body(x_ref, o_ref):
    X = x_ref[...]
    A = _remap(X[0:128, :].T)
    Bp = jnp.concatenate(
        [_remap(X[128:L, :].T), jnp.zeros((IBR, 128 - (L - 128)), jnp.int32)],
        axis=1)
    o_ref[...] = jnp.stack([A, Bp], axis=1).reshape(2 * IBR, 128)


_tr_idx = pl.pallas_call(
    _tr_idx_body,
    grid=(B // IBR,),
    in_specs=[pl.BlockSpec((L, IBR), lambda i: (0, i))],
    out_specs=pl.BlockSpec((2 * IBR, 128), lambda i: (i, 0)),
    out_shape=jax.ShapeDtypeStruct((2 * B, 128), jnp.int32),
)


def kernel(inlets, weight):
    wlin = _tr_table(weight.T)
    idx = _tr_idx(inlets[0].T)
    return _embbag(idx, wlin.reshape(4 * VPAD, D))


# speedup vs baseline: 1.4333x; 1.2228x over previous
import functools

import jax
import jax.numpy as jnp
from jax import lax
from jax.experimental import pallas as pl
from jax.experimental.pallas import tpu as pltpu
from jax.experimental.pallas import tpu_sc as plsc

B = 16384
L = 200
D = 32
NV = 1000000
NW = 32
BPW = B // NW
CB = 4
NCHUNKS = BPW // CB

TBC = 4096
TGRID = (NV + TBC - 1) // TBC
VPAD = TGRID * TBC

_mesh = plsc.VectorSubcoreMesh(core_axis_name="c", subcore_axis_name="s")


@functools.partial(
    pl.kernel,
    mesh=_mesh,
    out_type=jax.ShapeDtypeStruct((B, D), jnp.float32),
    scratch_types=[
        pltpu.VMEM((2, 2 * CB, 128), jnp.int32),
        pltpu.VMEM((2, CB * L, D), jnp.float32),
        pltpu.VMEM((BPW, D), jnp.float32),
        pltpu.SemaphoreType.DMA((2,)),
    ],
    compiler_params=pltpu.CompilerParams(use_tc_tiling_on_sc=False),
)
def _embbag(idx_hbm, table_hbm, out_hbm, idx_v, rows_v, out_v, sems):
    wid = lax.axis_index("s") * 2 + lax.axis_index("c")
    bag0 = wid * BPW
    scale = jnp.full((16,), 1.0 / L, jnp.float32)

    def fire(c, p):
        bb = bag0 + c * CB
        pltpu.sync_copy(idx_hbm.at[pl.ds(2 * bb, 2 * CB)], idx_v.at[p])
        for k in range(CB):
            pltpu.async_copy(
                table_hbm.at[idx_v.at[p, 2 * k]],
                rows_v.at[p, pl.ds(k * L, 128)],
                sems.at[p],
            )
            pltpu.async_copy(
                table_hbm.at[idx_v.at[p, 2 * k + 1, pl.ds(0, L - 128)]],
                rows_v.at[p, pl.ds(k * L + 128, L - 128)],
                sems.at[p],
            )

    def wait(p):
        pltpu.make_async_copy(
            table_hbm.at[pl.ds(0, CB * L)], rows_v.at[p], sems.at[p]
        ).wait()

    def reduce(c, p):
        for k in range(CB):
            base = k * L

            def red(j, accs):
                acc = list(accs)
                r = base + j * 8
                for u in range(8):
                    acc[u % 4] = acc[u % 4] + rows_v[p, r + u, 0:16]
                    acc[4 + u % 4] = acc[4 + u % 4] + rows_v[p, r + u, 16:32]
                return tuple(acc)

            z = jnp.zeros((16,), jnp.float32)
            accs = lax.fori_loop(0, L // 8, red, (z,) * 8)
            a_lo = (accs[0] + accs[1]) + (accs[2] + accs[3])
            a_hi = (accs[4] + accs[5]) + (accs[6] + accs[7])
            slot = c * CB + k
            out_v[slot, 0:16] = a_lo * scale
            out_v[slot, 16:32] = a_hi * scale

    fire(0, 0)

    def body(g, _):
        c0 = g * 2
        fire(c0 + 1, 1)
        wait(0)
        reduce(c0, 0)
        fire(c0 + 2, 0)
        wait(1)
        reduce(c0 + 1, 1)
        return 0

    lax.fori_loop(0, NCHUNKS // 2 - 1, body, 0)
    c0 = NCHUNKS - 2
    fire(c0 + 1, 1)
    wait(0)
    reduce(c0, 0)
    wait(1)
    reduce(c0 + 1, 1)

    pltpu.sync_copy(out_v, out_hbm.at[pl.ds(bag0, BPW)])


def _tr_table_body(w_ref, out_ref):
    out_ref[:, 0:32] = w_ref[...].T


_tr_table = pl.pallas_call(
    _tr_table_body,
    grid=(TGRID,),
    in_specs=[pl.BlockSpec((32, TBC), lambda i: (0, i))],
    out_specs=pl.BlockSpec((TBC, 128), lambda i: (i, 0)),
    out_shape=jax.ShapeDtypeStruct((VPAD, 128), jnp.float32),
)

IBR = 2048


def _remap(bv):
    return bv << 2


def _tr_idx_body(x_ref, o_ref):
    X = x_ref[...]
    A = _remap(X[0:128, :].T)
    Bp = jnp.concatenate(
        [_remap(X[128:L, :].T), jnp.zeros((IBR, 128 - (L - 128)), jnp.int32)],
        axis=1)
    o_ref[...] = jnp.stack([A, Bp], axis=1).reshape(2 * IBR, 128)


_tr_idx = pl.pallas_call(
    _tr_idx_body,
    grid=(B // IBR,),
    in_specs=[pl.BlockSpec((L, IBR), lambda i: (0, i))],
    out_specs=pl.BlockSpec((2 * IBR, 128), lambda i: (i, 0)),
    out_shape=jax.ShapeDtypeStruct((2 * B, 128), jnp.int32),
)


def kernel(inlets, weight):
    wlin = _tr_table(weight.T)
    idx = _tr_idx(inlets[0].T)
    return _embbag(idx, wlin.reshape(4 * VPAD, D))


# speedup vs baseline: 1.6264x; 1.1347x over previous
import functools

import jax
import jax.numpy as jnp
from jax import lax
from jax.experimental import pallas as pl
from jax.experimental.pallas import tpu as pltpu
from jax.experimental.pallas import tpu_sc as plsc

B = 16384
L = 200
D = 32
NV = 1000000
NW = 32
BPW = B // NW
CB = 4
NCHUNKS = BPW // CB

TBC = 8192
TGRID = (NV + TBC - 1) // TBC
VPAD = TGRID * TBC

_mesh = plsc.VectorSubcoreMesh(core_axis_name="c", subcore_axis_name="s")


@functools.partial(
    pl.kernel,
    mesh=_mesh,
    out_type=jax.ShapeDtypeStruct((B, D), jnp.float32),
    scratch_types=[
        pltpu.VMEM((2, 2 * CB, 128), jnp.int32),
        pltpu.VMEM((2, CB * L, D), jnp.float32),
        pltpu.VMEM((BPW, D), jnp.float32),
        pltpu.SemaphoreType.DMA((2,)),
    ],
    compiler_params=pltpu.CompilerParams(use_tc_tiling_on_sc=False),
)
def _embbag(idx_hbm, table_hbm, out_hbm, idx_v, rows_v, out_v, sems):
    wid = lax.axis_index("s") * 2 + lax.axis_index("c")
    bag0 = wid * BPW
    scale = jnp.full((16,), 1.0 / L, jnp.float32)

    def fire(c, p):
        bb = bag0 + c * CB
        pltpu.sync_copy(idx_hbm.at[pl.ds(2 * bb, 2 * CB)], idx_v.at[p])
        for k in range(CB):
            pltpu.async_copy(
                table_hbm.at[idx_v.at[p, 2 * k]],
                rows_v.at[p, pl.ds(k * L, 128)],
                sems.at[p],
            )
            pltpu.async_copy(
                table_hbm.at[idx_v.at[p, 2 * k + 1, pl.ds(0, L - 128)]],
                rows_v.at[p, pl.ds(k * L + 128, L - 128)],
                sems.at[p],
            )

    def wait(p):
        pltpu.make_async_copy(
            table_hbm.at[pl.ds(0, CB * L)], rows_v.at[p], sems.at[p]
        ).wait()

    def reduce(c, p):
        for k in range(CB):
            base = k * L

            def red(j, accs):
                acc = list(accs)
                r = base + j * 8
                for u in range(8):
                    acc[u % 4] = acc[u % 4] + rows_v[p, r + u, 0:16]
                    acc[4 + u % 4] = acc[4 + u % 4] + rows_v[p, r + u, 16:32]
                return tuple(acc)

            z = jnp.zeros((16,), jnp.float32)
            accs = lax.fori_loop(0, L // 8, red, (z,) * 8)
            a_lo = (accs[0] + accs[1]) + (accs[2] + accs[3])
            a_hi = (accs[4] + accs[5]) + (accs[6] + accs[7])
            slot = c * CB + k
            out_v[slot, 0:16] = a_lo * scale
            out_v[slot, 16:32] = a_hi * scale

    fire(0, 0)

    def body(g, _):
        c0 = g * 2
        fire(c0 + 1, 1)
        wait(0)
        reduce(c0, 0)
        fire(c0 + 2, 0)
        wait(1)
        reduce(c0 + 1, 1)
        return 0

    lax.fori_loop(0, NCHUNKS // 2 - 1, body, 0)
    c0 = NCHUNKS - 2
    fire(c0 + 1, 1)
    wait(0)
    reduce(c0, 0)
    wait(1)
    reduce(c0 + 1, 1)

    pltpu.sync_copy(out_v, out_hbm.at[pl.ds(bag0, BPW)])


def _tr_table_body(w_ref, out_ref):
    out_ref[:, 0:32] = w_ref[...].T


_tr_table = pl.pallas_call(
    _tr_table_body,
    grid=(TGRID,),
    in_specs=[pl.BlockSpec((32, TBC), lambda i: (0, i))],
    out_specs=pl.BlockSpec((TBC, 128), lambda i: (i, 0)),
    out_shape=jax.ShapeDtypeStruct((VPAD, 128), jnp.float32),
)

IBR = 2048


def _remap(bv):
    return bv << 2


def _tr_idx_body(x_ref, o_ref):
    X = x_ref[...]
    A = _remap(X[0:128, :].T)
    Bp = jnp.concatenate(
        [_remap(X[128:L, :].T), jnp.zeros((IBR, 128 - (L - 128)), jnp.int32)],
        axis=1)
    o_ref[...] = jnp.stack([A, Bp], axis=1).reshape(2 * IBR, 128)


_tr_idx = pl.pallas_call(
    _tr_idx_body,
    grid=(B // IBR,),
    in_specs=[pl.BlockSpec((L, IBR), lambda i: (0, i))],
    out_specs=pl.BlockSpec((2 * IBR, 128), lambda i: (i, 0)),
    out_shape=jax.ShapeDtypeStruct((2 * B, 128), jnp.int32),
)


def kernel(inlets, weight):
    wlin = _tr_table(weight.T)
    idx = _tr_idx(inlets[0].T)
    return _embbag(idx, wlin.reshape(4 * VPAD, D))


# speedup vs baseline: 1.7306x; 1.0641x over previous
import functools

import jax
import jax.numpy as jnp
from jax import lax
from jax.experimental import pallas as pl
from jax.experimental.pallas import tpu as pltpu
from jax.experimental.pallas import tpu_sc as plsc

B = 16384
L = 200
D = 32
NV = 1000000
NW = 32
BPW = B // NW
CB = 4
NCHUNKS = BPW // CB

TBC = 16384
TGRID = (NV + TBC - 1) // TBC
VPAD = TGRID * TBC

_mesh = plsc.VectorSubcoreMesh(core_axis_name="c", subcore_axis_name="s")


@functools.partial(
    pl.kernel,
    mesh=_mesh,
    out_type=jax.ShapeDtypeStruct((B, D), jnp.float32),
    scratch_types=[
        pltpu.VMEM((2, 2 * CB, 128), jnp.int32),
        pltpu.VMEM((2, CB * L, D), jnp.float32),
        pltpu.VMEM((BPW, D), jnp.float32),
        pltpu.SemaphoreType.DMA((2,)),
    ],
    compiler_params=pltpu.CompilerParams(use_tc_tiling_on_sc=False),
)
def _embbag(idx_hbm, table_hbm, out_hbm, idx_v, rows_v, out_v, sems):
    wid = lax.axis_index("s") * 2 + lax.axis_index("c")
    bag0 = wid * BPW
    scale = jnp.full((16,), 1.0 / L, jnp.float32)

    def fire(c, p):
        bb = bag0 + c * CB
        pltpu.sync_copy(idx_hbm.at[pl.ds(2 * bb, 2 * CB)], idx_v.at[p])
        for k in range(CB):
            pltpu.async_copy(
                table_hbm.at[idx_v.at[p, 2 * k]],
                rows_v.at[p, pl.ds(k * L, 128)],
                sems.at[p],
            )
            pltpu.async_copy(
                table_hbm.at[idx_v.at[p, 2 * k + 1, pl.ds(0, L - 128)]],
                rows_v.at[p, pl.ds(k * L + 128, L - 128)],
                sems.at[p],
            )

    def wait(p):
        pltpu.make_async_copy(
            table_hbm.at[pl.ds(0, CB * L)], rows_v.at[p], sems.at[p]
        ).wait()

    def reduce(c, p):
        for k in range(CB):
            base = k * L

            def red(j, accs):
                acc = list(accs)
                r = base + j * 8
                for u in range(8):
                    acc[u % 4] = acc[u % 4] + rows_v[p, r + u, 0:16]
                    acc[4 + u % 4] = acc[4 + u % 4] + rows_v[p, r + u, 16:32]
                return tuple(acc)

            z = jnp.zeros((16,), jnp.float32)
            accs = lax.fori_loop(0, L // 8, red, (z,) * 8)
            a_lo = (accs[0] + accs[1]) + (accs[2] + accs[3])
            a_hi = (accs[4] + accs[5]) + (accs[6] + accs[7])
            slot = c * CB + k
            out_v[slot, 0:16] = a_lo * scale
            out_v[slot, 16:32] = a_hi * scale

    fire(0, 0)

    def body(g, _):
        c0 = g * 2
        fire(c0 + 1, 1)
        wait(0)
        reduce(c0, 0)
        fire(c0 + 2, 0)
        wait(1)
        reduce(c0 + 1, 1)
        return 0

    lax.fori_loop(0, NCHUNKS // 2 - 1, body, 0)
    c0 = NCHUNKS - 2
    fire(c0 + 1, 1)
    wait(0)
    reduce(c0, 0)
    wait(1)
    reduce(c0 + 1, 1)

    pltpu.sync_copy(out_v, out_hbm.at[pl.ds(bag0, BPW)])


def _tr_table_body(w_ref, out_ref):
    out_ref[:, 0:32] = w_ref[...].T


_tr_table = pl.pallas_call(
    _tr_table_body,
    grid=(TGRID,),
    in_specs=[pl.BlockSpec((32, TBC), lambda i: (0, i))],
    out_specs=pl.BlockSpec((TBC, 128), lambda i: (i, 0)),
    out_shape=jax.ShapeDtypeStruct((VPAD, 128), jnp.float32),
)

IBR = 2048


def _remap(bv):
    return bv << 2


def _tr_idx_body(x_ref, o_ref):
    X = x_ref[...]
    A = _remap(X[0:128, :].T)
    Bp = jnp.concatenate(
        [_remap(X[128:L, :].T), jnp.zeros((IBR, 128 - (L - 128)), jnp.int32)],
        axis=1)
    o_ref[...] = jnp.stack([A, Bp], axis=1).reshape(2 * IBR, 128)


_tr_idx = pl.pallas_call(
    _tr_idx_body,
    grid=(B // IBR,),
    in_specs=[pl.BlockSpec((L, IBR), lambda i: (0, i))],
    out_specs=pl.BlockSpec((2 * IBR, 128), lambda i: (i, 0)),
    out_shape=jax.ShapeDtypeStruct((2 * B, 128), jnp.int32),
)


def kernel(inlets, weight):
    wlin = _tr_table(weight.T)
    idx = _tr_idx(inlets[0].T)
    return _embbag(idx, wlin.reshape(4 * VPAD, D))


# speedup vs baseline: 1.7491x; 1.0107x over previous
import functools

import jax
import jax.numpy as jnp
from jax import lax
from jax.experimental import pallas as pl
from jax.experimental.pallas import tpu as pltpu
from jax.experimental.pallas import tpu_sc as plsc

B = 16384
L = 200
D = 32
NV = 1000000
NW = 32
BPW = B // NW
CB = 4
NCHUNKS = BPW // CB

TBC = 32768
TGRID = (NV + TBC - 1) // TBC
VPAD = TGRID * TBC

_mesh = plsc.VectorSubcoreMesh(core_axis_name="c", subcore_axis_name="s")


@functools.partial(
    pl.kernel,
    mesh=_mesh,
    out_type=jax.ShapeDtypeStruct((B, D), jnp.float32),
    scratch_types=[
        pltpu.VMEM((2, 2 * CB, 128), jnp.int32),
        pltpu.VMEM((2, CB * L, D), jnp.float32),
        pltpu.VMEM((BPW, D), jnp.float32),
        pltpu.SemaphoreType.DMA((2,)),
    ],
    compiler_params=pltpu.CompilerParams(use_tc_tiling_on_sc=False),
)
def _embbag(idx_hbm, table_hbm, out_hbm, idx_v, rows_v, out_v, sems):
    wid = lax.axis_index("s") * 2 + lax.axis_index("c")
    bag0 = wid * BPW
    scale = jnp.full((16,), 1.0 / L, jnp.float32)

    def fire(c, p):
        bb = bag0 + c * CB
        pltpu.sync_copy(idx_hbm.at[pl.ds(2 * bb, 2 * CB)], idx_v.at[p])
        for k in range(CB):
            pltpu.async_copy(
                table_hbm.at[idx_v.at[p, 2 * k]],
                rows_v.at[p, pl.ds(k * L, 128)],
                sems.at[p],
            )
            pltpu.async_copy(
                table_hbm.at[idx_v.at[p, 2 * k + 1, pl.ds(0, L - 128)]],
                rows_v.at[p, pl.ds(k * L + 128, L - 128)],
                sems.at[p],
            )

    def wait(p):
        pltpu.make_async_copy(
            table_hbm.at[pl.ds(0, CB * L)], rows_v.at[p], sems.at[p]
        ).wait()

    def reduce(c, p):
        for k in range(CB):
            base = k * L

            def red(j, accs):
                acc = list(accs)
                r = base + j * 8
                for u in range(8):
                    acc[u % 4] = acc[u % 4] + rows_v[p, r + u, 0:16]
                    acc[4 + u % 4] = acc[4 + u % 4] + rows_v[p, r + u, 16:32]
                return tuple(acc)

            z = jnp.zeros((16,), jnp.float32)
            accs = lax.fori_loop(0, L // 8, red, (z,) * 8)
            a_lo = (accs[0] + accs[1]) + (accs[2] + accs[3])
            a_hi = (accs[4] + accs[5]) + (accs[6] + accs[7])
            slot = c * CB + k
            out_v[slot, 0:16] = a_lo * scale
            out_v[slot, 16:32] = a_hi * scale

    fire(0, 0)

    def body(g, _):
        c0 = g * 2
        fire(c0 + 1, 1)
        wait(0)
        reduce(c0, 0)
        fire(c0 + 2, 0)
        wait(1)
        reduce(c0 + 1, 1)
        return 0

    lax.fori_loop(0, NCHUNKS // 2 - 1, body, 0)
    c0 = NCHUNKS - 2
    fire(c0 + 1, 1)
    wait(0)
    reduce(c0, 0)
    wait(1)
    reduce(c0 + 1, 1)

    pltpu.sync_copy(out_v, out_hbm.at[pl.ds(bag0, BPW)])


def _tr_table_body(w_ref, out_ref):
    out_ref[:, 0:32] = w_ref[...].T


_tr_table = pl.pallas_call(
    _tr_table_body,
    grid=(TGRID,),
    in_specs=[pl.BlockSpec((32, TBC), lambda i: (0, i))],
    out_specs=pl.BlockSpec((TBC, 128), lambda i: (i, 0)),
    out_shape=jax.ShapeDtypeStruct((VPAD, 128), jnp.float32),
)

IBR = 2048


def _remap(bv):
    return bv << 2


def _tr_idx_body(x_ref, o_ref):
    X = x_ref[...]
    A = _remap(X[0:128, :].T)
    Bp = jnp.concatenate(
        [_remap(X[128:L, :].T), jnp.zeros((IBR, 128 - (L - 128)), jnp.int32)],
        axis=1)
    o_ref[...] = jnp.stack([A, Bp], axis=1).reshape(2 * IBR, 128)


_tr_idx = pl.pallas_call(
    _tr_idx_body,
    grid=(B // IBR,),
    in_specs=[pl.BlockSpec((L, IBR), lambda i: (0, i))],
    out_specs=pl.BlockSpec((2 * IBR, 128), lambda i: (i, 0)),
    out_shape=jax.ShapeDtypeStruct((2 * B, 128), jnp.int32),
)


def kernel(inlets, weight):
    wlin = _tr_table(weight.T)
    idx = _tr_idx(inlets[0].T)
    return _embbag(idx, wlin.reshape(4 * VPAD, D))
